# Initial kernel scaffold; baseline (speedup 1.0000x reference)
import jax
import jax.numpy as jnp
from jax.experimental import pallas as pl


def kernel(x, dim, index, value):
    raise NotImplementedError("write your pallas kernel here")


# speedup vs baseline: 3.4070x; 3.4070x over previous
import functools

import jax
import jax.numpy as jnp
from jax import lax
from jax.experimental import pallas as pl
from jax.experimental.pallas import tpu as pltpu
from jax.experimental.pallas import tpu_sc as plsc

NC = 2
NS = 16
NW = NC * NS
L = 16

CH = 128
FIRE = 8


def _copy_body(x_ref, o_ref):
    o_ref[...] = x_ref[...]


def _lin_body(dim_ref, idx_ref, o_ref, *, d):
    col = lax.broadcasted_iota(jnp.int32, idx_ref.shape, 1) % d
    o_ref[...] = (idx_ref[...] + dim_ref[0]) * d + col


def _scatter_body(out_hbm, lin_hbm, val_hbm, idx_v, val_v, sem, *, n_ch):
    wid = lax.axis_index("s") * NC + lax.axis_index("c")
    pltpu.sync_copy(lin_hbm.at[wid], idx_v)
    pltpu.sync_copy(val_hbm, val_v)

    @pl.loop(0, n_ch, step=FIRE)
    def _(j):
        cps = [
            pltpu.make_async_copy(val_v, out_hbm.at[idx_v.at[j + k]], sem)
            for k in range(FIRE)
        ]
        for c in cps:
            c.start()
        for c in cps:
            c.wait()


def kernel(x, dim, index, value):
    m, d = x.shape
    b = index.shape[0]
    md = m * d
    nidx = b * d

    cols = 128
    rows = md // cols
    blk = next(
        rows // g
        for g in range(16, rows + 1)
        if rows % g == 0 and (rows // g) % 8 == 0
    )
    xf = x.reshape(rows, cols)
    out_f = pl.pallas_call(
        _copy_body,
        grid=(rows // blk,),
        in_specs=[pl.BlockSpec((blk, cols), lambda i: (i, 0))],
        out_specs=pl.BlockSpec((blk, cols), lambda i: (i, 0)),
        out_shape=jax.ShapeDtypeStruct((rows, cols), jnp.float32),
    )(xf)

    icols = 128
    irows = nidx // icols
    dim_arr = jnp.asarray(dim, jnp.int32).reshape(1)
    lin = pl.pallas_call(
        functools.partial(_lin_body, d=d),
        in_specs=[
            pl.BlockSpec(memory_space=pltpu.SMEM),
            pl.BlockSpec((irows, icols), lambda: (0, 0)),
        ],
        out_specs=pl.BlockSpec((irows, icols), lambda: (0, 0)),
        out_shape=jax.ShapeDtypeStruct((irows, icols), jnp.int32),
    )(dim_arr, index.reshape(irows, icols))

    per_w = nidx // NW
    n_ch = per_w // CH
    lin3 = lin.reshape(NW, n_ch, CH)
    vals = jnp.full((CH,), value, jnp.float32)

    out_ref = jax.new_ref(out_f.reshape(md))
    mesh = plsc.VectorSubcoreMesh(
        core_axis_name="c", subcore_axis_name="s", num_cores=NC, num_subcores=NS
    )
    scatter = pl.kernel(
        functools.partial(_scatter_body, n_ch=n_ch),
        out_type=(),
        mesh=mesh,
        scratch_types=[
            pltpu.VMEM((n_ch, CH), jnp.int32),
            pltpu.VMEM((CH,), jnp.float32),
            pltpu.SemaphoreType.DMA,
        ],
    )
    scatter(out_ref, lin3, vals)
    return out_ref[...].reshape(m, d)
